# speedup vs baseline: 8.4186x; 8.4186x over previous
import functools

import jax
import jax.numpy as jnp
from jax import lax
from jax.experimental import pallas as pl
from jax.experimental.pallas import tpu as pltpu
from jax.experimental.pallas import tpu_sc as plsc

N_VOCAB = 100000
D_MODEL = 128
B_ROWS = 1024
B_COLS = 200
B_TOTAL = B_ROWS * B_COLS

NUM_WORKERS = 32
PER_WORKER = B_TOTAL // NUM_WORKERS
CHUNK = 128
NCHUNK = PER_WORKER // CHUNK
NBUF = 5
NGROUPS = NCHUNK // NBUF

_mesh = plsc.VectorSubcoreMesh(core_axis_name="c", subcore_axis_name="s")


@functools.partial(
    pl.kernel,
    out_type=jax.ShapeDtypeStruct((NUM_WORKERS, NCHUNK, CHUNK, D_MODEL),
                                  jnp.float32),
    mesh=_mesh,
    scratch_types=(
        [pltpu.VMEM((NCHUNK, CHUNK), jnp.int32),
         pltpu.VMEM((NBUF, CHUNK, D_MODEL), jnp.float32)]
        + [pltpu.SemaphoreType.DMA] * (2 * NBUF)
    ),
)
def _embed_sc(idx_hbm, wte_hbm, out_hbm, idx_v, bufs, *sems):
  gsem = sems[:NBUF]
  ssem = sems[NBUF:]
  wid = lax.axis_index("s") * 2 + lax.axis_index("c")

  pltpu.sync_copy(idx_hbm.at[wid], idx_v)

  def start_gather(j, b):
    pltpu.async_copy(wte_hbm.at[idx_v.at[j]], bufs.at[b], gsem[b])

  def wait_gather(b):
    pltpu.make_async_copy(wte_hbm.at[idx_v.at[0]], bufs.at[b],
                          gsem[b]).wait()

  def start_store(j, b):
    pltpu.async_copy(bufs.at[b], out_hbm.at[wid, j], ssem[b])

  def wait_store(j, b):
    pltpu.make_async_copy(bufs.at[b], out_hbm.at[wid, j], ssem[b]).wait()

  for b in range(NBUF):
    start_gather(b, b)

  def group_body(g, carry):
    for b in range(NBUF):
      j = g * NBUF + b
      wait_gather(b)
      start_store(j, b)
      wait_store(j, b)
      start_gather(j + NBUF, b)
    return carry

  lax.fori_loop(0, NGROUPS - 1, group_body, 0)

  for b in range(NBUF):
    j = (NGROUPS - 1) * NBUF + b
    wait_gather(b)
    start_store(j, b)
    wait_store(j, b)


def kernel(input_ids, wte):
  idx = input_ids.astype(jnp.int32).reshape(NUM_WORKERS, NCHUNK, CHUNK)
  out = _embed_sc(idx, wte)
  return out.reshape(B_ROWS, B_COLS, D_MODEL)
